# baseline (speedup 1.0000x reference)
import functools

import jax
import jax.numpy as jnp
from jax import lax
from jax.experimental import pallas as pl
from jax.experimental.pallas import tpu as pltpu
from jax.experimental.pallas import tpu_sc as plsc

N_NODES = 10000
N_EDGES = 320000
D = 128

_info = plsc.get_sparse_core_info()
_NC, _NS, _L = _info.num_cores, _info.num_subcores, _info.num_lanes
_NW = _NC * _NS
_EPW = ((N_EDGES + _NW - 1) // _NW + 127) // 128 * 128


def _proj_body(x_ref, w_ref, o_ref):
    o_ref[...] = jnp.dot(x_ref[...], w_ref[...],
                         preferred_element_type=jnp.float32)


def _project(X, Wc):
    return pl.pallas_call(
        _proj_body,
        out_shape=jax.ShapeDtypeStruct((N_NODES, 2), jnp.float32),
    )(X, Wc)


@functools.partial(
    pl.kernel,
    out_type=jax.ShapeDtypeStruct((N_EDGES, 1), jnp.float32),
    mesh=plsc.VectorSubcoreMesh(core_axis_name="c", subcore_axis_name="s"),
    compiler_params=pltpu.CompilerParams(needs_layout_passes=False,
                                         use_tc_tiling_on_sc=False),
    scratch_types=[
        pltpu.VMEM((2 * N_NODES,), jnp.float32),
        pltpu.VMEM((2, _EPW), jnp.int32),
        pltpu.VMEM((_EPW, 1), jnp.float32),
        pltpu.VMEM_SHARED((2 * N_NODES,), jnp.float32),
        pltpu.SemaphoreType.DMA,
    ],
)
def _sc_gather_add(y_hbm, edge_hbm, out_hbm, y_v, e_v, out_v, y_sp, sem):
    wid = lax.axis_index("s") * _NC + lax.axis_index("c")
    base = jnp.minimum(wid * _EPW, N_EDGES - _EPW)
    ce = pltpu.async_copy(edge_hbm.at[:, pl.ds(base, _EPW)], e_v, sem)
    @pl.when(lax.axis_index("s") == 0)
    def _():
        pltpu.sync_copy(y_hbm, y_sp)

    plsc.subcore_barrier()
    pltpu.sync_copy(y_sp, y_v)
    ce.wait()

    zeros = jnp.zeros((_L,), jnp.int32)
    iota = lax.iota(jnp.int32, _L)

    @plsc.parallel_loop(0, _EPW, step=_L, unroll=8)
    def _body(off):
        s = e_v[0, pl.ds(off, _L)]
        d = e_v[1, pl.ds(off, _L)]
        a = plsc.load_gather(y_v, [s * 2])
        b = plsc.load_gather(y_v, [d * 2 + 1])
        plsc.store_scatter(out_v, [off + iota, zeros], a + b)

    pltpu.sync_copy(out_v, out_hbm.at[pl.ds(base, _EPW), :])


def kernel(X, edge_index, W1, W2):
    Wc = jnp.concatenate([W1, W2], axis=1)
    y = _project(X, Wc)
    return _sc_gather_add(y.reshape(-1), edge_index)


# speedup vs baseline: 5.6811x; 5.6811x over previous
import functools

import jax
import jax.numpy as jnp
from jax import lax
from jax.experimental import pallas as pl
from jax.experimental.pallas import tpu as pltpu
from jax.experimental.pallas import tpu_sc as plsc

N_NODES = 10000
N_EDGES = 320000
D = 128

_info = plsc.get_sparse_core_info()
_NC, _NS, _L = _info.num_cores, _info.num_subcores, _info.num_lanes
_NW = _NC * _NS
_EPW = ((N_EDGES + _NW - 1) // _NW + 127) // 128 * 128


def _proj_body(x_ref, w_ref, o_ref):
    o_ref[...] = jnp.dot(x_ref[...], w_ref[...],
                         preferred_element_type=jnp.float32)


def _project(X, Wc):
    return pl.pallas_call(
        _proj_body,
        out_shape=jax.ShapeDtypeStruct((N_NODES, 2), jnp.float32),
    )(X, Wc)


@functools.partial(
    pl.kernel,
    out_type=jax.ShapeDtypeStruct((N_EDGES,), jnp.float32),
    mesh=plsc.VectorSubcoreMesh(core_axis_name="c", subcore_axis_name="s"),
    compiler_params=pltpu.CompilerParams(needs_layout_passes=False),
    scratch_types=[
        pltpu.VMEM((2 * N_NODES,), jnp.float32),
        pltpu.VMEM((2, _EPW), jnp.int32),
        pltpu.VMEM((_EPW,), jnp.float32),
        pltpu.VMEM_SHARED((2 * N_NODES,), jnp.float32),
        pltpu.SemaphoreType.DMA,
    ],
)
def _sc_gather_add(y_hbm, edge_hbm, out_hbm, y_v, e_v, out_v, y_sp, sem):
    wid = lax.axis_index("s") * _NC + lax.axis_index("c")
    base = jnp.minimum(wid * _EPW, N_EDGES - _EPW)
    ce = pltpu.async_copy(edge_hbm.at[:, pl.ds(base, _EPW)], e_v, sem)
    @pl.when(lax.axis_index("s") == 0)
    def _():
        pltpu.sync_copy(y_hbm, y_sp)

    plsc.subcore_barrier()
    pltpu.sync_copy(y_sp, y_v)
    ce.wait()

    @plsc.parallel_loop(0, _EPW, step=_L, unroll=8)
    def _body(off):
        s = e_v[0, pl.ds(off, _L)]
        d = e_v[1, pl.ds(off, _L)]
        a = plsc.load_gather(y_v, [s * 2])
        b = plsc.load_gather(y_v, [d * 2 + 1])
        out_v[pl.ds(off, _L)] = a + b

    pltpu.sync_copy(out_v, out_hbm.at[pl.ds(base, _EPW)])


def kernel(X, edge_index, W1, W2):
    Wc = jnp.concatenate([W1, W2], axis=1)
    y = _project(X, Wc)
    out = _sc_gather_add(y.reshape(-1), edge_index)
    return out[:, None]


# speedup vs baseline: 5.6870x; 1.0010x over previous
import functools

import jax
import jax.numpy as jnp
from jax import lax
from jax.experimental import pallas as pl
from jax.experimental.pallas import tpu as pltpu
from jax.experimental.pallas import tpu_sc as plsc

N_NODES = 10000
N_EDGES = 320000
D = 128

_info = plsc.get_sparse_core_info()
_NC, _NS, _L = _info.num_cores, _info.num_subcores, _info.num_lanes
_NW = _NC * _NS
_EPW = ((N_EDGES + _NW - 1) // _NW + 127) // 128 * 128


def _proj_body(x_ref, w_ref, o_ref):
    o_ref[...] = jnp.dot(x_ref[...], w_ref[...],
                         preferred_element_type=jnp.float32)


def _project(X, Wc):
    return pl.pallas_call(
        _proj_body,
        out_shape=jax.ShapeDtypeStruct((N_NODES, 2), jnp.float32),
    )(X, Wc)


@functools.partial(
    pl.kernel,
    out_type=jax.ShapeDtypeStruct((N_EDGES,), jnp.float32),
    mesh=plsc.VectorSubcoreMesh(core_axis_name="c", subcore_axis_name="s"),
    compiler_params=pltpu.CompilerParams(needs_layout_passes=False),
    scratch_types=[
        pltpu.VMEM((2 * N_NODES,), jnp.float32),
        pltpu.VMEM((2, _EPW), jnp.int32),
        pltpu.VMEM((_EPW,), jnp.float32),
        pltpu.VMEM_SHARED((2 * N_NODES,), jnp.float32),
        pltpu.SemaphoreType.DMA,
    ],
)
def _sc_gather_add(y_hbm, edge_hbm, out_hbm, y_v, e_v, out_v, y_sp, sem):
    wid = lax.axis_index("s") * _NC + lax.axis_index("c")
    base = jnp.minimum(wid * _EPW, N_EDGES - _EPW)
    ce = pltpu.async_copy(edge_hbm.at[:, pl.ds(base, _EPW)], e_v, sem)
    @pl.when(lax.axis_index("s") == 0)
    def _():
        pltpu.sync_copy(y_hbm, y_sp)

    plsc.subcore_barrier()
    pltpu.sync_copy(y_sp, y_v)
    ce.wait()

    @plsc.parallel_loop(0, _EPW, step=_L, unroll=4)
    def _body(off):
        s = e_v[0, pl.ds(off, _L)]
        d = e_v[1, pl.ds(off, _L)]
        a = plsc.load_gather(y_v, [s * 2])
        b = plsc.load_gather(y_v, [d * 2 + 1])
        out_v[pl.ds(off, _L)] = a + b

    pltpu.sync_copy(out_v, out_hbm.at[pl.ds(base, _EPW)])


def kernel(X, edge_index, W1, W2):
    Wc = jnp.concatenate([W1, W2], axis=1)
    y = _project(X, Wc)
    out = _sc_gather_add(y.reshape(-1), edge_index)
    return out[:, None]
